# Initial kernel scaffold; baseline (speedup 1.0000x reference)
import jax
import jax.numpy as jnp
from jax.experimental import pallas as pl


def kernel(x, x_mark, x_mask, y, y_mark, y_mask, params):
    raise NotImplementedError("write your pallas kernel here")


# speedup vs baseline: 1.2282x; 1.2282x over previous
import functools

import jax
import jax.numpy as jnp
import numpy as np
from jax.experimental import pallas as pl
from jax.experimental.pallas import tpu as pltpu

B, T, P, D = 8, 128, 32, 32
K = 64
H = 4
NL = 2
L = T + P
E = L * D
DH = K // H


def _selectors():
    j = np.arange(K)
    d = np.arange(D)
    rows = (d[:, None, None] * K + j[None, :, None])
    cols_d = np.arange(D * H)
    s_ct = ((rows // K) == (cols_d // H)) & ((j[None, :, None] // DH) == (cols_d % H))
    s_ct = s_ct.reshape(D * K, D * H).astype(np.float32)
    cols_t = np.arange(H * D)
    s_tc = ((rows // K) == (cols_t % D)) & ((j[None, :, None] // DH) == (cols_t // D))
    s_tc = s_tc.reshape(D * K, H * D).astype(np.float32)
    return s_ct, s_ct.T.copy(), s_tc, s_tc.T.copy()


_dot = functools.partial(jnp.dot, precision=jax.lax.Precision.HIGHEST)


def _ln(g, b, v):
    mu = jnp.mean(v, axis=-1, keepdims=True)
    c = v - mu
    var = jnp.mean(c * c, axis=-1, keepdims=True)
    return c * jax.lax.rsqrt(var + 1e-5) * g + b


def _mab_ct(p, C_emb, T_emb, U2d, s_ct, s_ctT):
    q = _dot(C_emb, p["fc_q"]["w"]) + p["fc_q"]["b"]
    wk, wv = p["fc_k"]["w"], p["fc_v"]["w"]
    k3 = (_dot(U2d, wk[K:]).reshape(L, D, K)
          + (_dot(T_emb, wk[:K]) + p["fc_k"]["b"])[:, None, :])
    v3 = (_dot(U2d, wv[K:]).reshape(L, D, K)
          + (_dot(T_emb, wv[:K]) + p["fc_v"]["b"])[:, None, :])
    prod = (k3 * q[None, :, :]).reshape(L, D * K)
    s = _dot(prod, s_ct) * (1.0 / np.sqrt(K))
    s = s - jnp.max(s, axis=0, keepdims=True)
    e = jnp.exp(s)
    a = e / jnp.sum(e, axis=0, keepdims=True)
    a3 = _dot(a, s_ctT).reshape(L, D, K)
    o = q + jnp.sum(a3 * v3, axis=0)
    o = _ln(p["ln0_g"], p["ln0_b"], o)
    o = o + jax.nn.relu(_dot(o, p["fc_o"]["w"]) + p["fc_o"]["b"])
    return _ln(p["ln1_g"], p["ln1_b"], o)


def _mab_tc(p, T_emb, C_emb, U2d, s_tc, s_tcT):
    q = _dot(T_emb, p["fc_q"]["w"]) + p["fc_q"]["b"]
    wk, wv = p["fc_k"]["w"], p["fc_v"]["w"]
    k3 = (_dot(U2d, wk[K:]).reshape(L, D, K)
          + (_dot(C_emb, wk[:K]) + p["fc_k"]["b"])[None, :, :])
    v3 = (_dot(U2d, wv[K:]).reshape(L, D, K)
          + (_dot(C_emb, wv[:K]) + p["fc_v"]["b"])[None, :, :])
    prod = (k3 * q[:, None, :]).reshape(L, D * K)
    s = _dot(prod, s_tc) * (1.0 / np.sqrt(K))
    s4 = s.reshape(L, H, D)
    s4 = s4 - jnp.max(s4, axis=-1, keepdims=True)
    e4 = jnp.exp(s4)
    a = (e4 / jnp.sum(e4, axis=-1, keepdims=True)).reshape(L, H * D)
    a3 = _dot(a, s_tcT).reshape(L, D, K)
    o = q + jnp.sum(a3 * v3, axis=1)
    o = _ln(p["ln0_g"], p["ln0_b"], o)
    o = o + jax.nn.relu(_dot(o, p["fc_o"]["w"]) + p["fc_o"]["b"])
    return _ln(p["ln1_g"], p["ln1_b"], o)


def _encoder_body(treedef, n_params, marks_ref, uf_ref, tgtm_ref,
                  sct_ref, sctT_ref, stc_ref, stcT_ref, *refs):
    param_refs, out_ref = refs[:n_params], refs[n_params]
    prm = jax.tree.unflatten(treedef, [r[...] for r in param_refs])
    s_ct, s_ctT = sct_ref[...], sctT_ref[...]
    s_tc, s_tcT = stc_ref[...], stcT_ref[...]

    marks = marks_ref[0]
    uf = uf_ref[0]
    tgtm = tgtm_ref[0]

    T_emb = jax.nn.relu(marks * prm["time_init"]["w"] + prm["time_init"]["b"])
    C_emb = jax.nn.relu(prm["chan_init"]["w"] + prm["chan_init"]["b"])
    we = prm["edge_init"]["w"]
    U3 = jax.nn.relu(uf[:, :, None] * we[0].reshape(1, 1, K)
                     + tgtm[:, :, None] * we[1].reshape(1, 1, K)
                     + prm["edge_init"]["b"][None])
    U2d = U3.reshape(E, K)

    for lp in prm["layers"]:
        C_emb = _mab_ct(lp["ct_attn"], C_emb, T_emb, U2d, s_ct, s_ctT)
        T_emb = _mab_tc(lp["tc_attn"], T_emb, C_emb, U2d, s_tc, s_tcT)
        wn, bn = lp["edge_nn"]["w"], lp["edge_nn"]["b"]
        cat3 = (_dot(U2d, wn[:K]).reshape(L, D, K)
                + _dot(T_emb, wn[K:2 * K])[:, None, :]
                + (_dot(C_emb, wn[2 * K:]) + bn)[None, :, :])
        U2d = (jax.nn.relu(cat3).reshape(E, K) + U2d)

    wo, bo = prm["output"]["w"], prm["output"]["b"]
    u_term = jnp.sum(U2d.reshape(L, D, K) * wo[:K, 0][None, None, :], axis=-1)
    t_term = jnp.sum(T_emb * wo[K:2 * K, 0][None, :], axis=-1, keepdims=True)
    c_term = jnp.sum(C_emb * wo[2 * K:, 0][None, :], axis=-1, keepdims=True)
    out_ref[0] = u_term + t_term + c_term.reshape(1, D) + bo[0, 0]


def kernel(x, x_mark, x_mask, y, y_mark, y_mask, params):
    marks = jnp.concatenate([x_mark[:, :, 0], y_mark[:, :, 0]], axis=1)[:, :, None]
    xv = x * x_mask
    uf = jnp.concatenate([xv, jnp.zeros((B, P, D), jnp.float32)], axis=1)
    tgtm = jnp.concatenate([jnp.zeros((B, T, D), jnp.float32), y_mask], axis=1)

    params2 = jax.tree.map(lambda a: a.reshape(1, -1) if a.ndim == 1 else a, params)
    leaves, treedef = jax.tree.flatten(params2)
    s_ct, s_ctT, s_tc, s_tcT = (jnp.asarray(m) for m in _selectors())

    body = functools.partial(_encoder_body, treedef, len(leaves))
    bs_param = [pl.BlockSpec(l.shape, lambda b: (0,) * l.ndim) for l in leaves]
    out = pl.pallas_call(
        body,
        grid=(B,),
        in_specs=[
            pl.BlockSpec((1, L, 1), lambda b: (b, 0, 0)),
            pl.BlockSpec((1, L, D), lambda b: (b, 0, 0)),
            pl.BlockSpec((1, L, D), lambda b: (b, 0, 0)),
            pl.BlockSpec((D * K, D * H), lambda b: (0, 0)),
            pl.BlockSpec((D * H, D * K), lambda b: (0, 0)),
            pl.BlockSpec((D * K, H * D), lambda b: (0, 0)),
            pl.BlockSpec((H * D, D * K), lambda b: (0, 0)),
        ] + bs_param,
        out_specs=pl.BlockSpec((1, L, D), lambda b: (b, 0, 0)),
        out_shape=jax.ShapeDtypeStruct((B, L, D), jnp.float32),
        compiler_params=pltpu.CompilerParams(
            dimension_semantics=("arbitrary",),
        ),
    )(marks, uf, tgtm, s_ct, s_ctT, s_tc, s_tcT, *leaves)

    pred = out.reshape(B, E)
    tgt_u = jnp.concatenate([jnp.zeros((B, T, D), jnp.float32), y], axis=1).reshape(B, E)
    tgt_m = tgtm.reshape(B, E)
    return pred, tgt_u, tgt_m


# speedup vs baseline: 1.4716x; 1.1982x over previous
import functools

import jax
import jax.numpy as jnp
import numpy as np
from jax.experimental import pallas as pl
from jax.experimental.pallas import tpu as pltpu

B, T, P, D = 8, 128, 32, 32
K = 64
H = 4
NL = 2
L = T + P
E = L * D
DH = K // H
BP = 2

_dot = functools.partial(jnp.dot, precision=jax.lax.Precision.HIGHEST)


def _selectors():
    j = np.arange(K)
    d = np.arange(D)
    rows = (d[:, None, None] * K + j[None, :, None])
    cols_d = np.arange(D * H)
    s_ct = ((rows // K) == (cols_d // H)) & ((j[None, :, None] // DH) == (cols_d % H))
    s_ct = s_ct.reshape(D * K, D * H).astype(np.float32)
    cols_t = np.arange(H * D)
    s_tc = ((rows // K) == (cols_t % D)) & ((j[None, :, None] // DH) == (cols_t // D))
    s_tc = s_tc.reshape(D * K, H * D).astype(np.float32)
    return s_ct, s_ct.T.copy(), s_tc, s_tc.T.copy()


def _ln(g, b, v):
    mu = jnp.mean(v, axis=-1, keepdims=True)
    c = v - mu
    var = jnp.mean(c * c, axis=-1, keepdims=True)
    return c * jax.lax.rsqrt(var + 1e-5) * g + b


def _mab_tail(p, q2d, o2d):
    o = _ln(p["ln0_g"], p["ln0_b"], q2d + o2d)
    o = o + jax.nn.relu(_dot(o, p["fc_o"]["w"]) + p["fc_o"]["b"])
    return _ln(p["ln1_g"], p["ln1_b"], o)


def _mab_ct(p, C2d, T2d, ku4, vu4, s_ct, s_ctT):
    q2d = _dot(C2d, p["fc_q"]["w"]) + p["fc_q"]["b"]
    kt = _dot(T2d, p["fc_k"]["w"][:K]) + p["fc_k"]["b"]
    vt = _dot(T2d, p["fc_v"]["w"][:K]) + p["fc_v"]["b"]
    k4 = ku4 + kt.reshape(BP, L, 1, K)
    v4 = vu4 + vt.reshape(BP, L, 1, K)
    p_ = k4 * q2d.reshape(BP, 1, D, K)
    s = _dot(p_.reshape(BP * L, D * K), s_ct).reshape(BP, L, D * H)
    s = s * (1.0 / np.sqrt(K))
    s = s - jnp.max(s, axis=1, keepdims=True)
    e = jnp.exp(s)
    a = e / jnp.sum(e, axis=1, keepdims=True)
    a4 = _dot(a.reshape(BP * L, D * H), s_ctT).reshape(BP, L, D, K)
    o = jnp.sum(a4 * v4, axis=1)
    return _mab_tail(p, q2d, o.reshape(BP * D, K))


def _mab_tc(p, T2d, C2d, ku4, vu4, s_tc, s_tcT):
    q2d = _dot(T2d, p["fc_q"]["w"]) + p["fc_q"]["b"]
    kc = _dot(C2d, p["fc_k"]["w"][:K]) + p["fc_k"]["b"]
    vc = _dot(C2d, p["fc_v"]["w"][:K]) + p["fc_v"]["b"]
    k4 = ku4 + kc.reshape(BP, 1, D, K)
    v4 = vu4 + vc.reshape(BP, 1, D, K)
    p_ = k4 * q2d.reshape(BP, L, 1, K)
    s = _dot(p_.reshape(BP * L, D * K), s_tc) * (1.0 / np.sqrt(K))
    s3 = s.reshape(BP * L, H, D)
    s3 = s3 - jnp.max(s3, axis=-1, keepdims=True)
    e3 = jnp.exp(s3)
    a = (e3 / jnp.sum(e3, axis=-1, keepdims=True)).reshape(BP * L, H * D)
    a4 = _dot(a, s_tcT).reshape(BP, L, D, K)
    o = jnp.sum(a4 * v4, axis=2)
    return _mab_tail(p, q2d, o.reshape(BP * L, K))


def _encoder_body(treedef, n_params, marks_ref, uf_ref, tgtm_ref,
                  sct_ref, sctT_ref, stc_ref, stcT_ref, *refs):
    param_refs, out_ref = refs[:n_params], refs[n_params]
    prm = jax.tree.unflatten(treedef, [r[...] for r in param_refs])
    s_ct, s_ctT = sct_ref[...], sctT_ref[...]
    s_tc, s_tcT = stc_ref[...], stcT_ref[...]

    marks = marks_ref[...]
    uf = uf_ref[...]
    tgtm = tgtm_ref[...]

    T2d = jax.nn.relu(marks * prm["time_init"]["w"]
                      + prm["time_init"]["b"]).reshape(BP * L, K)
    C2d = jnp.tile(jax.nn.relu(prm["chan_init"]["w"] + prm["chan_init"]["b"]),
                   (BP, 1))
    we = prm["edge_init"]["w"]
    U2d = jax.nn.relu(uf[:, :, :, None] * we[0].reshape(1, 1, 1, K)
                      + tgtm[:, :, :, None] * we[1].reshape(1, 1, 1, K)
                      + prm["edge_init"]["b"].reshape(1, 1, 1, K)).reshape(BP * E, K)

    for lp in prm["layers"]:
        cp, tp = lp["ct_attn"], lp["tc_attn"]
        wkv = jnp.concatenate([cp["fc_k"]["w"][K:], cp["fc_v"]["w"][K:],
                               tp["fc_k"]["w"][K:], tp["fc_v"]["w"][K:]], axis=1)
        kv = _dot(U2d, wkv)
        kv4 = kv.reshape(BP, L, D, 4 * K)
        C2d = _mab_ct(cp, C2d, T2d, kv4[..., :K], kv4[..., K:2 * K], s_ct, s_ctT)
        T2d = _mab_tc(tp, T2d, C2d, kv4[..., 2 * K:3 * K], kv4[..., 3 * K:],
                      s_tc, s_tcT)
        wn, bn = lp["edge_nn"]["w"], lp["edge_nn"]["b"]
        cat4 = (_dot(U2d, wn[:K]).reshape(BP, L, D, K)
                + _dot(T2d, wn[K:2 * K]).reshape(BP, L, 1, K)
                + (_dot(C2d, wn[2 * K:]) + bn).reshape(BP, 1, D, K))
        U2d = jax.nn.relu(cat4).reshape(BP * E, K) + U2d

    wo, bo = prm["output"]["w"], prm["output"]["b"]
    u_term = jnp.sum(U2d.reshape(BP, L, D, K) * wo[:K, 0].reshape(1, 1, 1, K),
                     axis=-1)
    t_term = jnp.sum(T2d * wo[K:2 * K, 0][None, :], axis=-1).reshape(BP, L, 1)
    c_term = jnp.sum(C2d * wo[2 * K:, 0][None, :], axis=-1).reshape(BP, 1, D)
    out_ref[...] = u_term + t_term + c_term + bo[0, 0]


def kernel(x, x_mark, x_mask, y, y_mark, y_mask, params):
    marks = jnp.concatenate([x_mark[:, :, 0], y_mark[:, :, 0]], axis=1)[:, :, None]
    xv = x * x_mask
    uf = jnp.concatenate([xv, jnp.zeros((B, P, D), jnp.float32)], axis=1)
    tgtm = jnp.concatenate([jnp.zeros((B, T, D), jnp.float32), y_mask], axis=1)

    params2 = jax.tree.map(lambda a: a.reshape(1, -1) if a.ndim == 1 else a, params)
    leaves, treedef = jax.tree.flatten(params2)
    s_ct, s_ctT, s_tc, s_tcT = (jnp.asarray(m) for m in _selectors())

    body = functools.partial(_encoder_body, treedef, len(leaves))
    bs_param = [pl.BlockSpec(l.shape, lambda b: (0,) * l.ndim) for l in leaves]
    out = pl.pallas_call(
        body,
        grid=(B // BP,),
        in_specs=[
            pl.BlockSpec((BP, L, 1), lambda b: (b, 0, 0)),
            pl.BlockSpec((BP, L, D), lambda b: (b, 0, 0)),
            pl.BlockSpec((BP, L, D), lambda b: (b, 0, 0)),
            pl.BlockSpec((D * K, D * H), lambda b: (0, 0)),
            pl.BlockSpec((D * H, D * K), lambda b: (0, 0)),
            pl.BlockSpec((D * K, H * D), lambda b: (0, 0)),
            pl.BlockSpec((H * D, D * K), lambda b: (0, 0)),
        ] + bs_param,
        out_specs=pl.BlockSpec((BP, L, D), lambda b: (b, 0, 0)),
        out_shape=jax.ShapeDtypeStruct((B, L, D), jnp.float32),
        compiler_params=pltpu.CompilerParams(
            dimension_semantics=("arbitrary",),
            vmem_limit_bytes=110 * 1024 * 1024,
        ),
    )(marks, uf, tgtm, s_ct, s_ctT, s_tc, s_tcT, *leaves)

    pred = out.reshape(B, E)
    tgt_u = jnp.concatenate([jnp.zeros((B, T, D), jnp.float32), y], axis=1).reshape(B, E)
    tgt_m = tgtm.reshape(B, E)
    return pred, tgt_u, tgt_m


# speedup vs baseline: 1.7282x; 1.1743x over previous
import functools

import jax
import jax.numpy as jnp
import numpy as np
from jax.experimental import pallas as pl
from jax.experimental.pallas import tpu as pltpu

B, T, P, D = 8, 128, 32, 32
K = 64
H = 4
NL = 2
L = T + P
E = L * D
DH = K // H
BP = 2

_dot = functools.partial(jnp.dot, precision=jax.lax.Precision.HIGHEST)
_INV_SQRT_K = 1.0 / np.sqrt(K)


def _selectors():
    j = np.arange(K)
    d = np.arange(D)
    rows = (d[:, None, None] * K + j[None, :, None])
    cols_d = np.arange(D * H)
    s_ct = ((rows // K) == (cols_d // H)) & ((j[None, :, None] // DH) == (cols_d % H))
    s_ct = s_ct.reshape(D * K, D * H).astype(np.float32)
    cols_t = np.arange(H * D)
    s_tc = ((rows // K) == (cols_t % D)) & ((j[None, :, None] // DH) == (cols_t // D))
    s_tc = s_tc.reshape(D * K, H * D).astype(np.float32)
    hm = (j[None, :] // DH == np.arange(H)[:, None]).astype(np.float32)
    return s_ct, s_ct.T.copy(), s_tc, s_tc.T.copy(), hm


def _ln(g, b, v):
    mu = jnp.mean(v, axis=-1, keepdims=True)
    c = v - mu
    var = jnp.mean(c * c, axis=-1, keepdims=True)
    return c * jax.lax.rsqrt(var + 1e-5) * g + b


def _mab_tail(p, q2d, o2d):
    o = _ln(p["ln0_g"], p["ln0_b"], q2d + o2d)
    o = o + jax.nn.relu(_dot(o, p["fc_o"]["w"]) + p["fc_o"]["b"])
    return _ln(p["ln1_g"], p["ln1_b"], o)


def _mab_ct(p, C2d, T2d, ku4, vu4, s_ct, s_ctT, hm):
    q2d = _dot(C2d, p["fc_q"]["w"]) + p["fc_q"]["b"]
    kt = _dot(T2d, p["fc_k"]["w"][:K]) + p["fc_k"]["b"]
    vt = _dot(T2d, p["fc_v"]["w"][:K]) + p["fc_v"]["b"]
    q3 = q2d.reshape(BP, D, K)
    kt3 = kt.reshape(BP, L, K)
    vt3 = vt.reshape(BP, L, K)
    outs = []
    for b in range(BP):
        q_b = q3[b]
        qsel = (s_ct.reshape(D, K, D * H) * q_b[:, :, None]).reshape(D * K, D * H)
        s = _dot(ku4[b].reshape(L, D * K), qsel)
        qh = (q_b.T[:, :, None] * hm.T[:, None, :]).reshape(K, D * H)
        s = s + _dot(kt3[b], qh)
        s = s * _INV_SQRT_K
        s = s - jnp.max(s, axis=0, keepdims=True)
        e = jnp.exp(s)
        a = e / jnp.sum(e, axis=0, keepdims=True)
        a3 = _dot(a, s_ctT).reshape(L, D, K)
        o = jnp.sum(a3 * vu4[b], axis=0)
        m2 = _dot(a.T, vt3[b])
        o = o + jnp.sum(m2.reshape(D, H, K) * hm[None, :, :], axis=1)
        outs.append(o)
    o2d = jnp.concatenate(outs, axis=0)
    return _mab_tail(p, q2d, o2d)


def _mab_tc(p, T2d, C2d, ku4, vu4, s_tc, s_tcT, hm):
    q2d = _dot(T2d, p["fc_q"]["w"]) + p["fc_q"]["b"]
    kc = _dot(C2d, p["fc_k"]["w"][:K]) + p["fc_k"]["b"]
    vc = _dot(C2d, p["fc_v"]["w"][:K]) + p["fc_v"]["b"]
    q4 = q2d.reshape(BP, L, 1, K)
    prod = (ku4 * q4).reshape(BP * L, D * K)
    s = _dot(prod, s_tc)
    kc3 = kc.reshape(BP, D, K)
    vc3 = vc.reshape(BP, D, K)
    corr = []
    vch = []
    for b in range(BP):
        kch = (hm.T[:, :, None] * kc3[b].T[:, None, :]).reshape(K, H * D)
        corr.append(_dot(q2d.reshape(BP, L, K)[b], kch))
        vch.append((hm[:, None, :] * vc3[b][None, :, :]).reshape(H * D, K))
    s = (s + jnp.concatenate(corr, axis=0)) * _INV_SQRT_K
    s3 = s.reshape(BP * L, H, D)
    s3 = s3 - jnp.max(s3, axis=-1, keepdims=True)
    e3 = jnp.exp(s3)
    a = (e3 / jnp.sum(e3, axis=-1, keepdims=True)).reshape(BP * L, H * D)
    a4 = _dot(a, s_tcT).reshape(BP, L, D, K)
    o = jnp.sum(a4 * vu4, axis=2).reshape(BP * L, K)
    a3 = a.reshape(BP, L, H * D)
    o_corr = jnp.concatenate([_dot(a3[b], vch[b]) for b in range(BP)], axis=0)
    return _mab_tail(p, q2d, o + o_corr)


def _encoder_body(treedef, n_params, marks_ref, uf_ref, tgtm_ref,
                  sct_ref, sctT_ref, stc_ref, stcT_ref, hm_ref, *refs):
    param_refs, out_ref = refs[:n_params], refs[n_params]
    prm = jax.tree.unflatten(treedef, [r[...] for r in param_refs])
    s_ct, s_ctT = sct_ref[...], sctT_ref[...]
    s_tc, s_tcT = stc_ref[...], stcT_ref[...]
    hm = hm_ref[...]

    marks = marks_ref[...]
    uf = uf_ref[...]
    tgtm = tgtm_ref[...]

    T2d = jax.nn.relu(marks * prm["time_init"]["w"]
                      + prm["time_init"]["b"]).reshape(BP * L, K)
    C2d = jnp.tile(jax.nn.relu(prm["chan_init"]["w"] + prm["chan_init"]["b"]),
                   (BP, 1))
    we = prm["edge_init"]["w"]
    U2d = jax.nn.relu(uf[:, :, :, None] * we[0].reshape(1, 1, 1, K)
                      + tgtm[:, :, :, None] * we[1].reshape(1, 1, 1, K)
                      + prm["edge_init"]["b"].reshape(1, 1, 1, K)).reshape(BP * E, K)

    for lp in prm["layers"]:
        cp, tp, wn = lp["ct_attn"], lp["tc_attn"], lp["edge_nn"]["w"]
        wkv = jnp.concatenate([cp["fc_k"]["w"][K:], cp["fc_v"]["w"][K:],
                               tp["fc_k"]["w"][K:], tp["fc_v"]["w"][K:],
                               wn[:K]], axis=1)
        kv = _dot(U2d, wkv)
        kv4 = kv.reshape(BP, L, D, 5 * K)
        C2d = _mab_ct(cp, C2d, T2d, kv4[..., :K], kv4[..., K:2 * K],
                      s_ct, s_ctT, hm)
        T2d = _mab_tc(tp, T2d, C2d, kv4[..., 2 * K:3 * K], kv4[..., 3 * K:4 * K],
                      s_tc, s_tcT, hm)
        bn = lp["edge_nn"]["b"]
        cat4 = (kv4[..., 4 * K:]
                + _dot(T2d, wn[K:2 * K]).reshape(BP, L, 1, K)
                + (_dot(C2d, wn[2 * K:]) + bn).reshape(BP, 1, D, K))
        U2d = jax.nn.relu(cat4).reshape(BP * E, K) + U2d

    wo, bo = prm["output"]["w"], prm["output"]["b"]
    u_term = jnp.sum(U2d.reshape(BP, L, D, K) * wo[:K, 0].reshape(1, 1, 1, K),
                     axis=-1)
    t_term = jnp.sum(T2d * wo[K:2 * K, 0][None, :], axis=-1).reshape(BP, L, 1)
    c_term = jnp.sum(C2d * wo[2 * K:, 0][None, :], axis=-1).reshape(BP, 1, D)
    out_ref[...] = u_term + t_term + c_term + bo[0, 0]


def kernel(x, x_mark, x_mask, y, y_mark, y_mask, params):
    marks = jnp.concatenate([x_mark[:, :, 0], y_mark[:, :, 0]], axis=1)[:, :, None]
    xv = x * x_mask
    uf = jnp.concatenate([xv, jnp.zeros((B, P, D), jnp.float32)], axis=1)
    tgtm = jnp.concatenate([jnp.zeros((B, T, D), jnp.float32), y_mask], axis=1)

    params2 = jax.tree.map(lambda a: a.reshape(1, -1) if a.ndim == 1 else a, params)
    leaves, treedef = jax.tree.flatten(params2)
    s_ct, s_ctT, s_tc, s_tcT, hm = (jnp.asarray(m) for m in _selectors())

    body = functools.partial(_encoder_body, treedef, len(leaves))
    bs_param = [pl.BlockSpec(l.shape, lambda b: (0,) * l.ndim) for l in leaves]
    out = pl.pallas_call(
        body,
        grid=(B // BP,),
        in_specs=[
            pl.BlockSpec((BP, L, 1), lambda b: (b, 0, 0)),
            pl.BlockSpec((BP, L, D), lambda b: (b, 0, 0)),
            pl.BlockSpec((BP, L, D), lambda b: (b, 0, 0)),
            pl.BlockSpec((D * K, D * H), lambda b: (0, 0)),
            pl.BlockSpec((D * H, D * K), lambda b: (0, 0)),
            pl.BlockSpec((D * K, H * D), lambda b: (0, 0)),
            pl.BlockSpec((H * D, D * K), lambda b: (0, 0)),
            pl.BlockSpec((H, K), lambda b: (0, 0)),
        ] + bs_param,
        out_specs=pl.BlockSpec((BP, L, D), lambda b: (b, 0, 0)),
        out_shape=jax.ShapeDtypeStruct((B, L, D), jnp.float32),
        compiler_params=pltpu.CompilerParams(
            dimension_semantics=("arbitrary",),
            vmem_limit_bytes=110 * 1024 * 1024,
        ),
    )(marks, uf, tgtm, s_ct, s_ctT, s_tc, s_tcT, hm, *leaves)

    pred = out.reshape(B, E)
    tgt_u = jnp.concatenate([jnp.zeros((B, T, D), jnp.float32), y], axis=1).reshape(B, E)
    tgt_m = tgtm.reshape(B, E)
    return pred, tgt_u, tgt_m


# speedup vs baseline: 3.0477x; 1.7636x over previous
import functools

import jax
import jax.numpy as jnp
import numpy as np
from jax.experimental import pallas as pl
from jax.experimental.pallas import tpu as pltpu

B, T, P, D = 8, 128, 32, 32
K = 64
H = 4
NL = 2
L = T + P
E = L * D
DH = K // H
BP = 2

_dot = functools.partial(jnp.dot, precision=jax.lax.Precision.HIGHEST)
_fdot = jnp.dot
_INV_SQRT_K = 1.0 / np.sqrt(K)


def _selectors():
    j = np.arange(K)
    d = np.arange(D)
    rows = (d[:, None, None] * K + j[None, :, None])
    cols_d = np.arange(D * H)
    s_ct = ((rows // K) == (cols_d // H)) & ((j[None, :, None] // DH) == (cols_d % H))
    s_ct = s_ct.reshape(D * K, D * H).astype(np.float32)
    cols_t = np.arange(H * D)
    s_tc = ((rows // K) == (cols_t % D)) & ((j[None, :, None] // DH) == (cols_t // D))
    s_tc = s_tc.reshape(D * K, H * D).astype(np.float32)
    hm = (j[None, :] // DH == np.arange(H)[:, None]).astype(np.float32)
    return s_ct, s_ct.T.copy(), s_tc, s_tc.T.copy(), hm


def _ln(g, b, v):
    mu = jnp.mean(v, axis=-1, keepdims=True)
    c = v - mu
    var = jnp.mean(c * c, axis=-1, keepdims=True)
    return c * jax.lax.rsqrt(var + 1e-5) * g + b


def _mab_tail(p, q2d, o2d):
    o = _ln(p["ln0_g"], p["ln0_b"], q2d + o2d)
    o = o + jax.nn.relu(_dot(o, p["fc_o"]["w"]) + p["fc_o"]["b"])
    return _ln(p["ln1_g"], p["ln1_b"], o)


def _mab_ct(p, C2d, T2d, ku4, vu4, s_ct, s_ctT, hm):
    q2d = _dot(C2d, p["fc_q"]["w"]) + p["fc_q"]["b"]
    kt = _dot(T2d, p["fc_k"]["w"][:K]) + p["fc_k"]["b"]
    vt = _dot(T2d, p["fc_v"]["w"][:K]) + p["fc_v"]["b"]
    q3 = q2d.reshape(BP, D, K)
    kt3 = kt.reshape(BP, L, K)
    vt3 = vt.reshape(BP, L, K)
    outs = []
    for b in range(BP):
        q_b = q3[b]
        qsel = (s_ct.reshape(D, K, D * H) * q_b[:, :, None]).reshape(D * K, D * H)
        s = _fdot(ku4[b].reshape(L, D * K), qsel)
        qh = (q_b.T[:, :, None] * hm.T[:, None, :]).reshape(K, D * H)
        s = s + _dot(kt3[b], qh)
        s = s * _INV_SQRT_K
        s = s - jnp.max(s, axis=0, keepdims=True)
        e = jnp.exp(s)
        a = e / jnp.sum(e, axis=0, keepdims=True)
        a3 = _fdot(a, s_ctT).reshape(L, D, K)
        o = jnp.sum(a3 * vu4[b], axis=0)
        m2 = _dot(a.T, vt3[b])
        o = o + jnp.sum(m2.reshape(D, H, K) * hm[None, :, :], axis=1)
        outs.append(o)
    o2d = jnp.concatenate(outs, axis=0)
    return _mab_tail(p, q2d, o2d)


def _mab_tc(p, T2d, C2d, ku4, vu4, s_tc, s_tcT, hm):
    q2d = _dot(T2d, p["fc_q"]["w"]) + p["fc_q"]["b"]
    kc = _dot(C2d, p["fc_k"]["w"][:K]) + p["fc_k"]["b"]
    vc = _dot(C2d, p["fc_v"]["w"][:K]) + p["fc_v"]["b"]
    q4 = q2d.reshape(BP, L, 1, K)
    prod = (ku4 * q4).reshape(BP * L, D * K)
    s = _fdot(prod, s_tc)
    kc3 = kc.reshape(BP, D, K)
    vc3 = vc.reshape(BP, D, K)
    corr = []
    vch = []
    for b in range(BP):
        kch = (hm.T[:, :, None] * kc3[b].T[:, None, :]).reshape(K, H * D)
        corr.append(_dot(q2d.reshape(BP, L, K)[b], kch))
        vch.append((hm[:, None, :] * vc3[b][None, :, :]).reshape(H * D, K))
    s = (s + jnp.concatenate(corr, axis=0)) * _INV_SQRT_K
    s3 = s.reshape(BP * L, H, D)
    s3 = s3 - jnp.max(s3, axis=-1, keepdims=True)
    e3 = jnp.exp(s3)
    a = (e3 / jnp.sum(e3, axis=-1, keepdims=True)).reshape(BP * L, H * D)
    a4 = _fdot(a, s_tcT).reshape(BP, L, D, K)
    o = jnp.sum(a4 * vu4, axis=2).reshape(BP * L, K)
    a3 = a.reshape(BP, L, H * D)
    o_corr = jnp.concatenate([_dot(a3[b], vch[b]) for b in range(BP)], axis=0)
    return _mab_tail(p, q2d, o + o_corr)


def _encoder_body(treedef, n_params, marks_ref, uf_ref, tgtm_ref,
                  sct_ref, sctT_ref, stc_ref, stcT_ref, hm_ref, *refs):
    param_refs, out_ref = refs[:n_params], refs[n_params]
    prm = jax.tree.unflatten(treedef, [r[...] for r in param_refs])
    s_ct, s_ctT = sct_ref[...], sctT_ref[...]
    s_tc, s_tcT = stc_ref[...], stcT_ref[...]
    hm = hm_ref[...]

    marks = marks_ref[...]
    uf = uf_ref[...]
    tgtm = tgtm_ref[...]

    T2d = jax.nn.relu(marks * prm["time_init"]["w"]
                      + prm["time_init"]["b"]).reshape(BP * L, K)
    C2d = jnp.tile(jax.nn.relu(prm["chan_init"]["w"] + prm["chan_init"]["b"]),
                   (BP, 1))
    we = prm["edge_init"]["w"]
    U2d = jax.nn.relu(uf[:, :, :, None] * we[0].reshape(1, 1, 1, K)
                      + tgtm[:, :, :, None] * we[1].reshape(1, 1, 1, K)
                      + prm["edge_init"]["b"].reshape(1, 1, 1, K)).reshape(BP * E, K)

    for lp in prm["layers"]:
        cp, tp, wn = lp["ct_attn"], lp["tc_attn"], lp["edge_nn"]["w"]
        wkv = jnp.concatenate([cp["fc_k"]["w"][K:], cp["fc_v"]["w"][K:],
                               tp["fc_k"]["w"][K:], tp["fc_v"]["w"][K:],
                               wn[:K]], axis=1)
        kv = _fdot(U2d, wkv)
        kv4 = kv.reshape(BP, L, D, 5 * K)
        C2d = _mab_ct(cp, C2d, T2d, kv4[..., :K], kv4[..., K:2 * K],
                      s_ct, s_ctT, hm)
        T2d = _mab_tc(tp, T2d, C2d, kv4[..., 2 * K:3 * K], kv4[..., 3 * K:4 * K],
                      s_tc, s_tcT, hm)
        bn = lp["edge_nn"]["b"]
        cat4 = (kv4[..., 4 * K:]
                + _dot(T2d, wn[K:2 * K]).reshape(BP, L, 1, K)
                + (_dot(C2d, wn[2 * K:]) + bn).reshape(BP, 1, D, K))
        U2d = jax.nn.relu(cat4).reshape(BP * E, K) + U2d

    wo, bo = prm["output"]["w"], prm["output"]["b"]
    u_term = jnp.sum(U2d.reshape(BP, L, D, K) * wo[:K, 0].reshape(1, 1, 1, K),
                     axis=-1)
    t_term = jnp.sum(T2d * wo[K:2 * K, 0][None, :], axis=-1).reshape(BP, L, 1)
    c_term = jnp.sum(C2d * wo[2 * K:, 0][None, :], axis=-1).reshape(BP, 1, D)
    out_ref[...] = u_term + t_term + c_term + bo[0, 0]


def kernel(x, x_mark, x_mask, y, y_mark, y_mask, params):
    marks = jnp.concatenate([x_mark[:, :, 0], y_mark[:, :, 0]], axis=1)[:, :, None]
    xv = x * x_mask
    uf = jnp.concatenate([xv, jnp.zeros((B, P, D), jnp.float32)], axis=1)
    tgtm = jnp.concatenate([jnp.zeros((B, T, D), jnp.float32), y_mask], axis=1)

    params2 = jax.tree.map(lambda a: a.reshape(1, -1) if a.ndim == 1 else a, params)
    leaves, treedef = jax.tree.flatten(params2)
    s_ct, s_ctT, s_tc, s_tcT, hm = (jnp.asarray(m) for m in _selectors())

    body = functools.partial(_encoder_body, treedef, len(leaves))
    bs_param = [pl.BlockSpec(l.shape, lambda b: (0,) * l.ndim) for l in leaves]
    out = pl.pallas_call(
        body,
        grid=(B // BP,),
        in_specs=[
            pl.BlockSpec((BP, L, 1), lambda b: (b, 0, 0)),
            pl.BlockSpec((BP, L, D), lambda b: (b, 0, 0)),
            pl.BlockSpec((BP, L, D), lambda b: (b, 0, 0)),
            pl.BlockSpec((D * K, D * H), lambda b: (0, 0)),
            pl.BlockSpec((D * H, D * K), lambda b: (0, 0)),
            pl.BlockSpec((D * K, H * D), lambda b: (0, 0)),
            pl.BlockSpec((H * D, D * K), lambda b: (0, 0)),
            pl.BlockSpec((H, K), lambda b: (0, 0)),
        ] + bs_param,
        out_specs=pl.BlockSpec((BP, L, D), lambda b: (b, 0, 0)),
        out_shape=jax.ShapeDtypeStruct((B, L, D), jnp.float32),
        compiler_params=pltpu.CompilerParams(
            dimension_semantics=("arbitrary",),
            vmem_limit_bytes=110 * 1024 * 1024,
        ),
    )(marks, uf, tgtm, s_ct, s_ctT, s_tc, s_tcT, hm, *leaves)

    pred = out.reshape(B, E)
    tgt_u = jnp.concatenate([jnp.zeros((B, T, D), jnp.float32), y], axis=1).reshape(B, E)
    tgt_m = tgtm.reshape(B, E)
    return pred, tgt_u, tgt_m


# speedup vs baseline: 3.0503x; 1.0008x over previous
import functools

import jax
import jax.numpy as jnp
import numpy as np
from jax.experimental import pallas as pl
from jax.experimental.pallas import tpu as pltpu

B, T, P, D = 8, 128, 32, 32
K = 64
H = 4
NL = 2
L = T + P
E = L * D
DH = K // H
BP = 2

_dot = functools.partial(jnp.dot, precision=jax.lax.Precision.HIGHEST)
_fdot = jnp.dot
_INV_SQRT_K = 1.0 / np.sqrt(K)


def _selectors():
    j = np.arange(K)
    d = np.arange(D)
    rows = (d[:, None, None] * K + j[None, :, None])
    cols_d = np.arange(D * H)
    s_ct = ((rows // K) == (cols_d // H)) & ((j[None, :, None] // DH) == (cols_d % H))
    s_ct = s_ct.reshape(D * K, D * H).astype(np.float32)
    cols_t = np.arange(H * D)
    s_tc = ((rows // K) == (cols_t % D)) & ((j[None, :, None] // DH) == (cols_t // D))
    s_tc = s_tc.reshape(D * K, H * D).astype(np.float32)
    hm = (j[None, :] // DH == np.arange(H)[:, None]).astype(np.float32)
    return s_ct, s_ct.T.copy(), s_tc, s_tc.T.copy(), hm


def _ln(g, b, v):
    mu = jnp.mean(v, axis=-1, keepdims=True)
    c = v - mu
    var = jnp.mean(c * c, axis=-1, keepdims=True)
    return c * jax.lax.rsqrt(var + 1e-5) * g + b


def _mab_tail(p, q2d, o2d):
    o = _ln(p["ln0_g"], p["ln0_b"], q2d + o2d)
    o = o + jax.nn.relu(_dot(o, p["fc_o"]["w"]) + p["fc_o"]["b"])
    return _ln(p["ln1_g"], p["ln1_b"], o)


def _mab_ct(p, C2d, T2d, ku4, vu4, s_ct, s_ctT, hm):
    q2d = _dot(C2d, p["fc_q"]["w"]) + p["fc_q"]["b"]
    kt = _dot(T2d, p["fc_k"]["w"][:K]) + p["fc_k"]["b"]
    vt = _dot(T2d, p["fc_v"]["w"][:K]) + p["fc_v"]["b"]
    q3 = q2d.reshape(BP, D, K)
    kt3 = kt.reshape(BP, L, K)
    vt3 = vt.reshape(BP, L, K)
    outs = []
    for b in range(BP):
        q_b = q3[b]
        qsel = (s_ct.reshape(D, K, D * H) * q_b[:, :, None]).reshape(D * K, D * H)
        s = _fdot(ku4[b].reshape(L, D * K), qsel)
        qh = (q_b.T[:, :, None] * hm.T[:, None, :]).reshape(K, D * H)
        s = s + _dot(kt3[b], qh)
        s = s * _INV_SQRT_K
        s = s - jnp.max(s, axis=0, keepdims=True)
        e = jnp.exp(s)
        a = e / jnp.sum(e, axis=0, keepdims=True)
        a3 = _fdot(a, s_ctT).reshape(L, D, K)
        o = jnp.sum(a3 * vu4[b], axis=0)
        m2 = _dot(a.T, vt3[b])
        o = o + jnp.sum(m2.reshape(D, H, K) * hm[None, :, :], axis=1)
        outs.append(o)
    o2d = jnp.concatenate(outs, axis=0)
    return _mab_tail(p, q2d, o2d)


def _mab_tc(p, T2d, C2d, ku4, vu4, s_tc, s_tcT, hm):
    q2d = _dot(T2d, p["fc_q"]["w"]) + p["fc_q"]["b"]
    kc = _dot(C2d, p["fc_k"]["w"][:K]) + p["fc_k"]["b"]
    vc = _dot(C2d, p["fc_v"]["w"][:K]) + p["fc_v"]["b"]
    q4 = q2d.reshape(BP, L, 1, K)
    prod = (ku4 * q4).reshape(BP * L, D * K)
    s = _fdot(prod, s_tc)
    kc3 = kc.reshape(BP, D, K)
    vc3 = vc.reshape(BP, D, K)
    corr = []
    vch = []
    for b in range(BP):
        kch = (hm.T[:, :, None] * kc3[b].T[:, None, :]).reshape(K, H * D)
        corr.append(_dot(q2d.reshape(BP, L, K)[b], kch))
        vch.append((hm[:, None, :] * vc3[b][None, :, :]).reshape(H * D, K))
    s = (s + jnp.concatenate(corr, axis=0)) * _INV_SQRT_K
    s3 = s.reshape(BP * L, H, D)
    s3 = s3 - jnp.max(s3, axis=-1, keepdims=True)
    e3 = jnp.exp(s3)
    a = (e3 / jnp.sum(e3, axis=-1, keepdims=True)).reshape(BP * L, H * D)
    a4 = _fdot(a, s_tcT).reshape(BP, L, D, K)
    o = jnp.sum(a4 * vu4, axis=2).reshape(BP * L, K)
    a3 = a.reshape(BP, L, H * D)
    o_corr = jnp.concatenate([_dot(a3[b], vch[b]) for b in range(BP)], axis=0)
    return _mab_tail(p, q2d, o + o_corr)


def _encoder_body(treedef, n_params, marks_ref, uf_ref, tgtm_ref,
                  sct_ref, sctT_ref, stc_ref, stcT_ref, hm_ref, *refs):
    param_refs, out_ref = refs[:n_params], refs[n_params]
    prm = jax.tree.unflatten(treedef, [r[...] for r in param_refs])
    s_ct, s_ctT = sct_ref[...], sctT_ref[...]
    s_tc, s_tcT = stc_ref[...], stcT_ref[...]
    hm = hm_ref[...]

    marks = marks_ref[...]
    uf = uf_ref[...]
    tgtm = tgtm_ref[...]

    T2d = jax.nn.relu(marks * prm["time_init"]["w"]
                      + prm["time_init"]["b"]).reshape(BP * L, K)
    C2d = jnp.tile(jax.nn.relu(prm["chan_init"]["w"] + prm["chan_init"]["b"]),
                   (BP, 1))
    we = prm["edge_init"]["w"]
    U2d = jax.nn.relu(uf[:, :, :, None] * we[0].reshape(1, 1, 1, K)
                      + tgtm[:, :, :, None] * we[1].reshape(1, 1, 1, K)
                      + prm["edge_init"]["b"].reshape(1, 1, 1, K)).reshape(BP * E, K)

    for lp in prm["layers"]:
        cp, tp, wn = lp["ct_attn"], lp["tc_attn"], lp["edge_nn"]["w"]
        wkv = jnp.concatenate([cp["fc_k"]["w"][K:], cp["fc_v"]["w"][K:],
                               tp["fc_k"]["w"][K:], tp["fc_v"]["w"][K:],
                               wn[:K]], axis=1)
        kv = _fdot(U2d, wkv)
        kv4 = kv.reshape(BP, L, D, 5 * K)
        C2d = _mab_ct(cp, C2d, T2d, kv4[..., :K], kv4[..., K:2 * K],
                      s_ct, s_ctT, hm)
        T2d = _mab_tc(tp, T2d, C2d, kv4[..., 2 * K:3 * K], kv4[..., 3 * K:4 * K],
                      s_tc, s_tcT, hm)
        bn = lp["edge_nn"]["b"]
        cat4 = (kv4[..., 4 * K:]
                + _dot(T2d, wn[K:2 * K]).reshape(BP, L, 1, K)
                + (_dot(C2d, wn[2 * K:]) + bn).reshape(BP, 1, D, K))
        U2d = jax.nn.relu(cat4).reshape(BP * E, K) + U2d

    wo, bo = prm["output"]["w"], prm["output"]["b"]
    u_term = jnp.sum(U2d.reshape(BP, L, D, K) * wo[:K, 0].reshape(1, 1, 1, K),
                     axis=-1)
    t_term = jnp.sum(T2d * wo[K:2 * K, 0][None, :], axis=-1).reshape(BP, L, 1)
    c_term = jnp.sum(C2d * wo[2 * K:, 0][None, :], axis=-1).reshape(BP, 1, D)
    out_ref[...] = u_term + t_term + c_term + bo[0, 0]


def kernel(x, x_mark, x_mask, y, y_mark, y_mask, params):
    marks = jnp.concatenate([x_mark[:, :, 0], y_mark[:, :, 0]], axis=1)[:, :, None]
    xv = x * x_mask
    uf = jnp.concatenate([xv, jnp.zeros((B, P, D), jnp.float32)], axis=1)
    tgtm = jnp.concatenate([jnp.zeros((B, T, D), jnp.float32), y_mask], axis=1)

    params2 = jax.tree.map(lambda a: a.reshape(1, -1) if a.ndim == 1 else a, params)
    leaves, treedef = jax.tree.flatten(params2)
    s_ct, s_ctT, s_tc, s_tcT, hm = (jnp.asarray(m) for m in _selectors())

    body = functools.partial(_encoder_body, treedef, len(leaves))
    bs_param = [pl.BlockSpec(l.shape, lambda b: (0,) * l.ndim) for l in leaves]
    out = pl.pallas_call(
        body,
        grid=(B // BP,),
        in_specs=[
            pl.BlockSpec((BP, L, 1), lambda b: (b, 0, 0)),
            pl.BlockSpec((BP, L, D), lambda b: (b, 0, 0)),
            pl.BlockSpec((BP, L, D), lambda b: (b, 0, 0)),
            pl.BlockSpec((D * K, D * H), lambda b: (0, 0)),
            pl.BlockSpec((D * H, D * K), lambda b: (0, 0)),
            pl.BlockSpec((D * K, H * D), lambda b: (0, 0)),
            pl.BlockSpec((H * D, D * K), lambda b: (0, 0)),
            pl.BlockSpec((H, K), lambda b: (0, 0)),
        ] + bs_param,
        out_specs=pl.BlockSpec((BP, L, D), lambda b: (b, 0, 0)),
        out_shape=jax.ShapeDtypeStruct((B, L, D), jnp.float32),
        compiler_params=pltpu.CompilerParams(
            dimension_semantics=("parallel",),
            vmem_limit_bytes=110 * 1024 * 1024,
        ),
    )(marks, uf, tgtm, s_ct, s_ctT, s_tc, s_tcT, hm, *leaves)

    pred = out.reshape(B, E)
    tgt_u = jnp.concatenate([jnp.zeros((B, T, D), jnp.float32), y], axis=1).reshape(B, E)
    tgt_m = tgtm.reshape(B, E)
    return pred, tgt_u, tgt_m
